# baseline (speedup 1.0000x reference)
import functools

import jax
import jax.numpy as jnp
from jax import lax
from jax.experimental import pallas as pl
from jax.experimental.pallas import tpu as pltpu
from jax.experimental.pallas import tpu_sc as plsc

_NC = 2
_NS = 16
_NW = _NC * _NS
_EPS = 1e-05


_NSLOT = 4


def _make_sc_gather(n, d, chunk):
    b_per_w = n // _NW
    n_chunks = b_per_w // chunk
    assert b_per_w % chunk == 0 and n_chunks % _NSLOT == 0
    mesh = plsc.VectorSubcoreMesh(
        core_axis_name="c", subcore_axis_name="s",
        num_cores=_NC, num_subcores=_NS)

    @functools.partial(
        pl.kernel,
        out_type=jax.ShapeDtypeStruct((n, d), jnp.float32),
        mesh=mesh,
        scratch_types=[
            pltpu.VMEM((b_per_w,), jnp.int32),
        ]
        + [pltpu.VMEM((chunk, d), jnp.float32)] * _NSLOT
        + [pltpu.SemaphoreType.DMA] * (2 * _NSLOT),
        compiler_params=pltpu.CompilerParams(needs_layout_passes=False),
    )
    def run(idx_hbm, tab_hbm, out_hbm, idx_v, *bufsem):
        bufs = bufsem[:_NSLOT]
        gsem = bufsem[_NSLOT:2 * _NSLOT]
        ssem = bufsem[2 * _NSLOT:]
        wid = lax.axis_index("s") * _NC + lax.axis_index("c")
        base = wid * b_per_w
        pltpu.sync_copy(idx_hbm.at[pl.ds(base, b_per_w)], idx_v)

        def start_gather(c, p):
            pltpu.async_copy(
                tab_hbm.at[idx_v.at[pl.ds(c * chunk, chunk)]],
                bufs[p], gsem[p])

        def wait_gather(c, p):
            pltpu.make_async_copy(
                tab_hbm.at[idx_v.at[pl.ds(c * chunk, chunk)]],
                bufs[p], gsem[p]).wait()

        def start_store(c, p):
            pltpu.async_copy(
                bufs[p], out_hbm.at[pl.ds(base + c * chunk, chunk)], ssem[p])

        def wait_store(c, p):
            pltpu.make_async_copy(
                bufs[p], out_hbm.at[pl.ds(base + c * chunk, chunk)],
                ssem[p]).wait()

        for p in range(_NSLOT):
            start_gather(p, p)

        def step(c, p):
            wait_gather(c, p)
            start_store(c, p)

            @pl.when(c + _NSLOT < n_chunks)
            def _():
                wait_store(c, p)
                start_gather(c + _NSLOT, p)

        def ring_body(g, carry):
            for p in range(_NSLOT):
                step(g * _NSLOT + p, p)
            return carry

        lax.fori_loop(0, n_chunks // _NSLOT, ring_body, 0, unroll=False)

        for p in range(_NSLOT):
            wait_store(n_chunks - _NSLOT + p, p)

    return run


def _norm_body(w_ref, x_ref, acc_ref, o_ref, *, d):
    del acc_ref
    x = x_ref[...]
    ms = jnp.mean(x * x, axis=-1, keepdims=True)
    o_ref[...] = x * lax.rsqrt(ms + _EPS) * w_ref[...]


def _make_tc_norm_slice(n, nk, d, block_rows, row_off, aliased):
    assert nk % block_rows == 0 and row_off % block_rows == 0
    grid = (nk // block_rows,)
    blk_off = row_off // block_rows
    return pl.pallas_call(
        functools.partial(_norm_body, d=d),
        grid=grid,
        in_specs=[
            pl.BlockSpec((1, d), lambda i: (0, 0)),
            pl.BlockSpec((block_rows, d), lambda i: (i, 0)),
            pl.BlockSpec(memory_space=pl.ANY),
        ],
        out_specs=pl.BlockSpec((block_rows, d), lambda i: (i + blk_off, 0)),
        out_shape=jax.ShapeDtypeStruct((n, d), jnp.float32),
        input_output_aliases={2: 0} if aliased else {},
    )


def kernel(input_ids, table, rms_weight):
    batch, seq = input_ids.shape
    vocab, d = table.shape
    n = batch * seq
    nchunk = 2
    block_rows = 1024
    nk = n // nchunk
    idx = input_ids.reshape(n).astype(jnp.int32)
    w2d = rms_weight.astype(jnp.float32).reshape(1, d)

    sc_gather = _make_sc_gather(nk, d, chunk=16)
    gathered = [
        lax.optimization_barrier(sc_gather(idx[k * nk:(k + 1) * nk], table))
        for k in range(nchunk)
    ]

    out = _make_tc_norm_slice(n, nk, d, block_rows, 0, aliased=False)(
        w2d, gathered[0], gathered[0])
    for k in range(1, nchunk):
        out = _make_tc_norm_slice(n, nk, d, block_rows, k * nk, aliased=True)(
            w2d, gathered[k], out)
    return out.reshape(batch, seq, d)


# speedup vs baseline: 1.0202x; 1.0202x over previous
import functools

import jax
import jax.numpy as jnp
from jax import lax
from jax.experimental import pallas as pl
from jax.experimental.pallas import tpu as pltpu
from jax.experimental.pallas import tpu_sc as plsc

_NC = 2
_NS = 16
_NW = _NC * _NS
_EPS = 1e-05


_NSLOT = 4


def _make_sc_gather(n, d, chunk):
    b_per_w = n // _NW
    n_chunks = b_per_w // chunk
    assert b_per_w % chunk == 0 and n_chunks % _NSLOT == 0
    mesh = plsc.VectorSubcoreMesh(
        core_axis_name="c", subcore_axis_name="s",
        num_cores=_NC, num_subcores=_NS)

    @functools.partial(
        pl.kernel,
        out_type=jax.ShapeDtypeStruct((n, d), jnp.float32),
        mesh=mesh,
        scratch_types=[
            pltpu.VMEM((b_per_w,), jnp.int32),
        ]
        + [pltpu.VMEM((chunk, d), jnp.float32)] * _NSLOT
        + [pltpu.SemaphoreType.DMA] * (2 * _NSLOT),
        compiler_params=pltpu.CompilerParams(needs_layout_passes=False),
    )
    def run(idx_hbm, tab_hbm, out_hbm, idx_v, *bufsem):
        bufs = bufsem[:_NSLOT]
        gsem = bufsem[_NSLOT:2 * _NSLOT]
        ssem = bufsem[2 * _NSLOT:]
        wid = lax.axis_index("s") * _NC + lax.axis_index("c")
        base = wid * b_per_w
        pltpu.sync_copy(idx_hbm.at[pl.ds(base, b_per_w)], idx_v)

        def start_gather(c, p):
            pltpu.async_copy(
                tab_hbm.at[idx_v.at[pl.ds(c * chunk, chunk)]],
                bufs[p], gsem[p])

        def wait_gather(c, p):
            pltpu.make_async_copy(
                tab_hbm.at[idx_v.at[pl.ds(c * chunk, chunk)]],
                bufs[p], gsem[p]).wait()

        def start_store(c, p):
            pltpu.async_copy(
                bufs[p], out_hbm.at[pl.ds(base + c * chunk, chunk)], ssem[p])

        def wait_store(c, p):
            pltpu.make_async_copy(
                bufs[p], out_hbm.at[pl.ds(base + c * chunk, chunk)],
                ssem[p]).wait()

        for p in range(_NSLOT):
            start_gather(p, p)

        def step(c, p):
            wait_gather(c, p)
            start_store(c, p)

            @pl.when(c + _NSLOT < n_chunks)
            def _():
                wait_store(c, p)
                start_gather(c + _NSLOT, p)

        def ring_body(g, carry):
            for p in range(_NSLOT):
                step(g * _NSLOT + p, p)
            return carry

        lax.fori_loop(0, n_chunks // _NSLOT, ring_body, 0, unroll=False)

        for p in range(_NSLOT):
            wait_store(n_chunks - _NSLOT + p, p)

    return run


def _norm_body(w_ref, x_ref, acc_ref, o_ref, *, d):
    del acc_ref
    x = x_ref[...]
    ms = jnp.mean(x * x, axis=-1, keepdims=True)
    o_ref[...] = x * lax.rsqrt(ms + _EPS) * w_ref[...]


def _make_tc_norm_slice(n, nk, d, block_rows, row_off, aliased):
    assert nk % block_rows == 0 and row_off % block_rows == 0
    grid = (nk // block_rows,)
    blk_off = row_off // block_rows
    return pl.pallas_call(
        functools.partial(_norm_body, d=d),
        grid=grid,
        in_specs=[
            pl.BlockSpec((1, d), lambda i: (0, 0)),
            pl.BlockSpec((block_rows, d), lambda i: (i, 0)),
            pl.BlockSpec(memory_space=pl.ANY),
        ],
        out_specs=pl.BlockSpec((block_rows, d), lambda i: (i + blk_off, 0)),
        out_shape=jax.ShapeDtypeStruct((n, d), jnp.float32),
        input_output_aliases={2: 0} if aliased else {},
    )


def kernel(input_ids, table, rms_weight):
    batch, seq = input_ids.shape
    vocab, d = table.shape
    n = batch * seq
    nchunk = 1
    block_rows = 2048
    nk = n // nchunk
    idx = input_ids.reshape(n).astype(jnp.int32)
    w2d = rms_weight.astype(jnp.float32).reshape(1, d)

    sc_gather = _make_sc_gather(nk, d, chunk=16)
    gathered = [
        lax.optimization_barrier(sc_gather(idx[k * nk:(k + 1) * nk], table))
        for k in range(nchunk)
    ]

    out = _make_tc_norm_slice(n, nk, d, block_rows, 0, aliased=False)(
        w2d, gathered[0], gathered[0])
    for k in range(1, nchunk):
        out = _make_tc_norm_slice(n, nk, d, block_rows, k * nk, aliased=True)(
            w2d, gathered[k], out)
    return out.reshape(batch, seq, d)
